# Initial kernel scaffold; baseline (speedup 1.0000x reference)
import functools

import jax
import jax.numpy as jnp
from jax import lax
from jax.experimental import pallas as pl
from jax.experimental.pallas import tpu as pltpu
from jax.experimental.pallas import tpu_sc as plsc

N = 10000
E = 320000
H = 128
G = 16
A = 18

EC = 128
NCHUNK = E // EC
NW = 32
CPW = -(-NCHUNK // NW)
NRC = N // EC
NREM = N - NRC * EC


def _pre_body(x_ref, w_ref, as_ref, ad_ref, h_ref, asv_ref, adv_ref, c_ref):
    h = jnp.dot(x_ref[...], w_ref[...], preferred_element_type=jnp.float32)
    h_ref[...] = h
    asv = jnp.sum(h * as_ref[...], axis=1, keepdims=True)
    adv = jnp.sum(h * ad_ref[...], axis=1, keepdims=True)
    asv_ref[...] = asv
    adv_ref[...] = adv
    m = jnp.max(asv) + jnp.max(adv)
    c_ref[...] = jnp.full((1, 1), 1.0, jnp.float32) * jnp.maximum(m, 0.2 * m)


def _tc_pre(x, w, a_s, a_d):
    return pl.pallas_call(
        _pre_body,
        out_shape=[
            jax.ShapeDtypeStruct((N, H), jnp.float32),
            jax.ShapeDtypeStruct((N, 1), jnp.float32),
            jax.ShapeDtypeStruct((N, 1), jnp.float32),
            jax.ShapeDtypeStruct((1, 1), jnp.float32),
        ],
    )(x, w, a_s, a_d)


def _combine(parts_ref, den0_ref, den1_ref, b_ref):
    hsum = parts_ref[0] + parts_ref[1]
    den = den0_ref[...] + den1_ref[...]
    return hsum / (den + 1e-16) + b_ref[...]


def _mid_body(parts_ref, den0_ref, den1_ref, b_ref, batr_ref, batc_ref,
              gw_ref, gb_ref, gms_ref, w_ref, as_ref, ad_ref,
              h_ref, asv_ref, adv_ref, c_ref):
    h = _combine(parts_ref, den0_ref, den1_ref, b_ref)
    gi = lax.broadcasted_iota(jnp.int32, (G, N), 0)
    oh = (gi == batr_ref[...]).astype(jnp.float32)
    gj = lax.broadcasted_iota(jnp.int32, (N, G), 1)
    ohT = (gj == batc_ref[...]).astype(jnp.float32)
    cnt = jnp.maximum(jnp.sum(oh, axis=1, keepdims=True), 1.0)
    mean = jnp.dot(oh, h, preferred_element_type=jnp.float32) / cnt
    xc = h - gms_ref[...] * jnp.dot(ohT, mean,
                                    preferred_element_type=jnp.float32)
    var = jnp.dot(oh, xc * xc, preferred_element_type=jnp.float32) / cnt
    hn = gw_ref[...] * xc / jnp.sqrt(
        jnp.dot(ohT, var, preferred_element_type=jnp.float32) + 1e-5
    ) + gb_ref[...]
    hn = jnp.maximum(hn, 0.01 * hn)
    h2 = jnp.dot(hn, w_ref[...], preferred_element_type=jnp.float32)
    h_ref[...] = h2
    asv = jnp.sum(h2 * as_ref[...], axis=1, keepdims=True)
    adv = jnp.sum(h2 * ad_ref[...], axis=1, keepdims=True)
    asv_ref[...] = asv
    adv_ref[...] = adv
    m = jnp.max(asv) + jnp.max(adv)
    c_ref[...] = jnp.full((1, 1), 1.0, jnp.float32) * jnp.maximum(m, 0.2 * m)


def _tc_mid(parts, den0, den1, b, batr, batc, gw, gb, gms, w, a_s, a_d):
    return pl.pallas_call(
        _mid_body,
        out_shape=[
            jax.ShapeDtypeStruct((N, H), jnp.float32),
            jax.ShapeDtypeStruct((N, 1), jnp.float32),
            jax.ShapeDtypeStruct((N, 1), jnp.float32),
            jax.ShapeDtypeStruct((1, 1), jnp.float32),
        ],
    )(parts, den0, den1, b, batr, batc, gw, gb, gms, w, a_s, a_d)


def _fin_body(parts_ref, den0_ref, den1_ref, b_ref, batr_ref,
              hw1_ref, hb1_ref, hw2_ref, hb2_ref, out_ref):
    h = _combine(parts_ref, den0_ref, den1_ref, b_ref)
    gi = lax.broadcasted_iota(jnp.int32, (G, N), 0)
    oh = (gi == batr_ref[...]).astype(jnp.float32)
    pooled = jnp.dot(oh, h, preferred_element_type=jnp.float32)
    z = jnp.dot(pooled, hw1_ref[...],
                preferred_element_type=jnp.float32) + hb1_ref[...]
    z = jnp.maximum(z, 0.01 * z)
    out_ref[...] = jnp.dot(z, hw2_ref[...],
                           preferred_element_type=jnp.float32) + hb2_ref[...]


def _tc_fin(parts, den0, den1, b, batr, hw1, hb1, hw2, hb2):
    return pl.pallas_call(
        _fin_body,
        out_shape=jax.ShapeDtypeStruct((G, A), jnp.float32),
    )(parts, den0, den1, b, batr, hw1, hb1, hw2, hb2)


_MESH = plsc.VectorSubcoreMesh(core_axis_name="c", subcore_axis_name="s")


@functools.partial(
    pl.kernel,
    out_type=[
        jax.ShapeDtypeStruct((2, N, H), jnp.float32),
        jax.ShapeDtypeStruct((2, N), jnp.float32),
    ],
    mesh=_MESH,
    scratch_types=[
        pltpu.VMEM_SHARED((N, H), jnp.float32),
        pltpu.VMEM_SHARED((N,), jnp.float32),
        pltpu.VMEM_SHARED((N,), jnp.float32),
        pltpu.VMEM_SHARED((N,), jnp.float32),
        pltpu.VMEM((EC,), jnp.int32),
        pltpu.VMEM((EC,), jnp.int32),
        pltpu.VMEM((EC,), jnp.float32),
        pltpu.VMEM((EC,), jnp.float32),
        pltpu.VMEM((EC,), jnp.float32),
        pltpu.VMEM((EC, H), jnp.float32),
        pltpu.VMEM((EC, H), jnp.float32),
        pltpu.VMEM((16,), jnp.float32),
    ],
)
def _sc_edge(h_hbm, asv_hbm, adv_hbm, src_hbm, dst_hbm, c_hbm,
             out_hbm, den_hbm,
             out_sh, den_sh, asv_sh, adv_sh,
             srcv, dstv, asg, adg, pv, rows, zrows, cv):
    c = lax.axis_index("c")
    s = lax.axis_index("s")
    wid = c * 16 + s

    zero16 = jnp.zeros((16,), jnp.float32)

    def zbody(i, carry):
        for k in range(H // 16):
            zrows[i, pl.ds(k * 16, 16)] = zero16
        return carry

    lax.fori_loop(0, EC, zbody, 0)

    @pl.when(s == 0)
    def _():
        pltpu.sync_copy(asv_hbm, asv_sh)
        pltpu.sync_copy(adv_hbm, adv_sh)

    pltpu.sync_copy(c_hbm, cv)

    for t in range(5):
        idx = s + 16 * t

        @pl.when(idx < NRC)
        def _():
            pltpu.sync_copy(zrows, out_sh.at[pl.ds(idx * EC, EC)])
            pltpu.sync_copy(zrows.at[0], den_sh.at[pl.ds(idx * EC, EC)])

        @pl.when(idx == NRC)
        def _():
            pltpu.sync_copy(zrows.at[pl.ds(0, NREM)],
                            out_sh.at[pl.ds(NRC * EC, NREM)])
            pltpu.sync_copy(zrows.at[0, pl.ds(0, NREM)],
                            den_sh.at[pl.ds(NRC * EC, NREM)])

    plsc.subcore_barrier()
    cvec = cv[...]

    def chunk_body(t, carry):
        chunk = wid + NW * t

        @pl.when(chunk < NCHUNK)
        def _():
            base = chunk * EC
            pltpu.sync_copy(src_hbm.at[pl.ds(base, EC)], srcv)
            pltpu.sync_copy(dst_hbm.at[pl.ds(base, EC)], dstv)
            pltpu.sync_copy(asv_sh.at[srcv], asg)
            pltpu.sync_copy(adv_sh.at[dstv], adg)
            pltpu.sync_copy(h_hbm.at[srcv], rows)
            for j in range(EC // 16):
                sl = pl.ds(j * 16, 16)
                e = asg[sl] + adg[sl]
                e = jnp.maximum(e, 0.2 * e)
                pv[sl] = jnp.exp(e - cvec)
            pltpu.sync_copy(pv, den_sh.at[dstv], add=True)

            def rbody(i, carry2):
                psc = pv[i]
                for k in range(H // 16):
                    slk = pl.ds(k * 16, 16)
                    rows[i, slk] = rows[i, slk] * psc
                return carry2

            lax.fori_loop(0, EC, rbody, 0)
            pltpu.sync_copy(rows, out_sh.at[dstv], add=True)

        return carry

    lax.fori_loop(0, CPW, chunk_body, 0)
    plsc.subcore_barrier()

    for t in range(5):
        idx = s + 16 * t

        @pl.when(idx < NRC)
        def _():
            pltpu.sync_copy(out_sh.at[pl.ds(idx * EC, EC)],
                            out_hbm.at[c, pl.ds(idx * EC, EC)])
            pltpu.sync_copy(den_sh.at[pl.ds(idx * EC, EC)],
                            den_hbm.at[c, pl.ds(idx * EC, EC)])

        @pl.when(idx == NRC)
        def _():
            pltpu.sync_copy(out_sh.at[pl.ds(NRC * EC, NREM)],
                            out_hbm.at[c, pl.ds(NRC * EC, NREM)])
            pltpu.sync_copy(den_sh.at[pl.ds(NRC * EC, NREM)],
                            den_hbm.at[c, pl.ds(NRC * EC, NREM)])


def kernel(x, edge_index, batch, W1, as1, ad1, b1, W2, as2, ad2, b2,
           W3, as3, ad3, b3, gn1_w, gn1_b, gn1_ms, gn2_w, gn2_b, gn2_ms,
           hW1, hb1, hW2, hb2):
    src = edge_index[0]
    dst = edge_index[1]
    batr = batch.reshape(1, N)
    batc = batch.reshape(N, 1)

    def edge(h, asv, adv, c1):
        c16 = jnp.broadcast_to(c1.reshape(()), (16,))
        parts, den = _sc_edge(h, asv.reshape(N), adv.reshape(N),
                              src, dst, c16)
        return parts, den[0].reshape(N, 1), den[1].reshape(N, 1)

    h1, asv1, adv1, c1 = _tc_pre(x, W1, as1.reshape(1, H), ad1.reshape(1, H))
    parts1, d10, d11 = edge(h1, asv1, adv1, c1)
    h2, asv2, adv2, c2 = _tc_mid(parts1, d10, d11, b1.reshape(1, H),
                                 batr, batc, gn1_w.reshape(1, H),
                                 gn1_b.reshape(1, H), gn1_ms.reshape(1, H),
                                 W2, as2.reshape(1, H), ad2.reshape(1, H))
    parts2, d20, d21 = edge(h2, asv2, adv2, c2)
    h3, asv3, adv3, c3 = _tc_mid(parts2, d20, d21, b2.reshape(1, H),
                                 batr, batc, gn2_w.reshape(1, H),
                                 gn2_b.reshape(1, H), gn2_ms.reshape(1, H),
                                 W3, as3.reshape(1, H), ad3.reshape(1, H))
    parts3, d30, d31 = edge(h3, asv3, adv3, c3)
    return _tc_fin(parts3, d30, d31, b3.reshape(1, H), batr,
                   hW1, hb1.reshape(1, H), hW2, hb2.reshape(1, A))


# speedup vs baseline: 26.9804x; 26.9804x over previous
import functools

import jax
import jax.numpy as jnp
from jax import lax
from jax.experimental import pallas as pl
from jax.experimental.pallas import tpu as pltpu
from jax.experimental.pallas import tpu_sc as plsc

N = 10000
E = 320000
H = 128
G = 16
A = 18

EC = 128
NCHUNK = E // EC
NW = 32
CPW = -(-NCHUNK // NW)
NP = 10112
NRC = NP // EC


def _pre_body(x_ref, w_ref, as_ref, ad_ref, h_ref, asv_ref, adv_ref, c_ref):
    h = jnp.dot(x_ref[...], w_ref[...], preferred_element_type=jnp.float32)
    h_ref[...] = h
    asv = jnp.sum(h * as_ref[...], axis=1, keepdims=True)
    adv = jnp.sum(h * ad_ref[...], axis=1, keepdims=True)
    asv_ref[...] = asv
    adv_ref[...] = adv
    m = jnp.max(asv) + jnp.max(adv)
    c_ref[...] = jnp.full((1, 1), 1.0, jnp.float32) * jnp.maximum(m, 0.2 * m)


def _tc_pre(x, w, a_s, a_d):
    return pl.pallas_call(
        _pre_body,
        out_shape=[
            jax.ShapeDtypeStruct((N, H), jnp.float32),
            jax.ShapeDtypeStruct((N, 1), jnp.float32),
            jax.ShapeDtypeStruct((N, 1), jnp.float32),
            jax.ShapeDtypeStruct((1, 1), jnp.float32),
        ],
    )(x, w, a_s, a_d)


def _combine(parts_ref, den0_ref, den1_ref, b_ref):
    hsum = parts_ref[0] + parts_ref[1]
    den = den0_ref[...] + den1_ref[...]
    return hsum / (den + 1e-16) + b_ref[...]


def _mid_body(parts_ref, den0_ref, den1_ref, b_ref, batr_ref, batc_ref,
              gw_ref, gb_ref, gms_ref, w_ref, as_ref, ad_ref,
              h_ref, asv_ref, adv_ref, c_ref):
    h = _combine(parts_ref, den0_ref, den1_ref, b_ref)
    gi = lax.broadcasted_iota(jnp.int32, (G, N), 0)
    oh = (gi == batr_ref[...]).astype(jnp.float32)
    gj = lax.broadcasted_iota(jnp.int32, (N, G), 1)
    ohT = (gj == batc_ref[...]).astype(jnp.float32)
    cnt = jnp.maximum(jnp.sum(oh, axis=1, keepdims=True), 1.0)
    mean = jnp.dot(oh, h, preferred_element_type=jnp.float32) / cnt
    xc = h - gms_ref[...] * jnp.dot(ohT, mean,
                                    preferred_element_type=jnp.float32)
    var = jnp.dot(oh, xc * xc, preferred_element_type=jnp.float32) / cnt
    hn = gw_ref[...] * xc / jnp.sqrt(
        jnp.dot(ohT, var, preferred_element_type=jnp.float32) + 1e-5
    ) + gb_ref[...]
    hn = jnp.maximum(hn, 0.01 * hn)
    h2 = jnp.dot(hn, w_ref[...], preferred_element_type=jnp.float32)
    h_ref[...] = h2
    asv = jnp.sum(h2 * as_ref[...], axis=1, keepdims=True)
    adv = jnp.sum(h2 * ad_ref[...], axis=1, keepdims=True)
    asv_ref[...] = asv
    adv_ref[...] = adv
    m = jnp.max(asv) + jnp.max(adv)
    c_ref[...] = jnp.full((1, 1), 1.0, jnp.float32) * jnp.maximum(m, 0.2 * m)


def _tc_mid(parts, den0, den1, b, batr, batc, gw, gb, gms, w, a_s, a_d):
    return pl.pallas_call(
        _mid_body,
        out_shape=[
            jax.ShapeDtypeStruct((N, H), jnp.float32),
            jax.ShapeDtypeStruct((N, 1), jnp.float32),
            jax.ShapeDtypeStruct((N, 1), jnp.float32),
            jax.ShapeDtypeStruct((1, 1), jnp.float32),
        ],
    )(parts, den0, den1, b, batr, batc, gw, gb, gms, w, a_s, a_d)


def _fin_body(parts_ref, den0_ref, den1_ref, b_ref, batr_ref,
              hw1_ref, hb1_ref, hw2_ref, hb2_ref, out_ref):
    h = _combine(parts_ref, den0_ref, den1_ref, b_ref)
    gi = lax.broadcasted_iota(jnp.int32, (G, N), 0)
    oh = (gi == batr_ref[...]).astype(jnp.float32)
    pooled = jnp.dot(oh, h, preferred_element_type=jnp.float32)
    z = jnp.dot(pooled, hw1_ref[...],
                preferred_element_type=jnp.float32) + hb1_ref[...]
    z = jnp.maximum(z, 0.01 * z)
    out_ref[...] = jnp.dot(z, hw2_ref[...],
                           preferred_element_type=jnp.float32) + hb2_ref[...]


def _tc_fin(parts, den0, den1, b, batr, hw1, hb1, hw2, hb2):
    return pl.pallas_call(
        _fin_body,
        out_shape=jax.ShapeDtypeStruct((G, A), jnp.float32),
    )(parts, den0, den1, b, batr, hw1, hb1, hw2, hb2)


_MESH = plsc.VectorSubcoreMesh(core_axis_name="c", subcore_axis_name="s")


@functools.partial(
    pl.kernel,
    out_type=[
        jax.ShapeDtypeStruct((2, NP, H), jnp.float32),
        jax.ShapeDtypeStruct((2, NP), jnp.float32),
    ],
    mesh=_MESH,
    scratch_types=[
        pltpu.VMEM_SHARED((NP, H), jnp.float32),
        pltpu.VMEM_SHARED((NP,), jnp.float32),
        pltpu.VMEM_SHARED((N,), jnp.float32),
        pltpu.VMEM_SHARED((N,), jnp.float32),
        pltpu.VMEM((EC,), jnp.int32),
        pltpu.VMEM((EC,), jnp.int32),
        pltpu.VMEM((EC,), jnp.float32),
        pltpu.VMEM((EC,), jnp.float32),
        pltpu.VMEM((EC,), jnp.float32),
        pltpu.VMEM((EC, H), jnp.float32),
        pltpu.VMEM((EC, H), jnp.float32),
        pltpu.VMEM((EC,), jnp.float32),
    ],
)
def _sc_edge(h_hbm, asv_hbm, adv_hbm, src_hbm, dst_hbm, c_hbm,
             out_hbm, den_hbm,
             out_sh, den_sh, asv_sh, adv_sh,
             srcv, dstv, asg, adg, pv, rows, zrows, cv):
    c = lax.axis_index("c")
    s = lax.axis_index("s")
    wid = c * 16 + s

    zero16 = jnp.zeros((16,), jnp.float32)

    def zbody(i, carry):
        for k in range(H // 16):
            zrows[i, pl.ds(k * 16, 16)] = zero16
        return carry

    lax.fori_loop(0, EC, zbody, 0)

    @pl.when(s == 0)
    def _():
        pltpu.sync_copy(asv_hbm, asv_sh)
        pltpu.sync_copy(adv_hbm, adv_sh)

    pltpu.sync_copy(c_hbm, cv)

    for t in range(5):
        idx = s + 16 * t

        @pl.when(idx < NRC)
        def _():
            pltpu.sync_copy(zrows, out_sh.at[pl.ds(idx * EC, EC)])
            pltpu.sync_copy(zrows.at[0], den_sh.at[pl.ds(idx * EC, EC)])

    plsc.subcore_barrier()
    cvec = cv[pl.ds(0, 16)]

    def chunk_body(t, carry):
        chunk = wid + NW * t

        @pl.when(chunk < NCHUNK)
        def _():
            base = chunk * EC
            pltpu.sync_copy(src_hbm.at[pl.ds(base, EC)], srcv)
            pltpu.sync_copy(dst_hbm.at[pl.ds(base, EC)], dstv)
            pltpu.sync_copy(asv_sh.at[srcv], asg)
            pltpu.sync_copy(adv_sh.at[dstv], adg)
            pltpu.sync_copy(h_hbm.at[srcv], rows)
            for j in range(EC // 16):
                sl = pl.ds(j * 16, 16)
                e = asg[sl] + adg[sl]
                e = jnp.maximum(e, 0.2 * e)
                pv[sl] = jnp.exp(e - cvec)
            pltpu.sync_copy(pv, den_sh.at[dstv], add=True)

            def rbody(jj, carry2):
                pvec = pv[pl.ds(jj * 16, 16)]
                for m in range(16):
                    psc = pvec[m]
                    i = jj * 16 + m
                    for k in range(H // 16):
                        slk = pl.ds(k * 16, 16)
                        rows[i, slk] = rows[i, slk] * psc
                return carry2

            lax.fori_loop(0, EC // 16, rbody, 0)
            pltpu.sync_copy(rows, out_sh.at[dstv], add=True)

        return carry

    lax.fori_loop(0, CPW, chunk_body, 0)
    plsc.subcore_barrier()

    for t in range(5):
        idx = s + 16 * t

        @pl.when(idx < NRC)
        def _():
            pltpu.sync_copy(out_sh.at[pl.ds(idx * EC, EC)],
                            out_hbm.at[c, pl.ds(idx * EC, EC)])
            pltpu.sync_copy(den_sh.at[pl.ds(idx * EC, EC)],
                            den_hbm.at[c, pl.ds(idx * EC, EC)])


def kernel(x, edge_index, batch, W1, as1, ad1, b1, W2, as2, ad2, b2,
           W3, as3, ad3, b3, gn1_w, gn1_b, gn1_ms, gn2_w, gn2_b, gn2_ms,
           hW1, hb1, hW2, hb2):
    src = edge_index[0]
    dst = edge_index[1]
    batr = batch.reshape(1, N)
    batc = batch.reshape(N, 1)

    def edge(h, asv, adv, c1):
        c128 = jnp.broadcast_to(c1.reshape(()), (EC,))
        parts, den = _sc_edge(h, asv.reshape(N), adv.reshape(N),
                              src, dst, c128)
        return (parts[:, :N, :], den[0, :N].reshape(N, 1),
                den[1, :N].reshape(N, 1))

    h1, asv1, adv1, c1 = _tc_pre(x, W1, as1.reshape(1, H), ad1.reshape(1, H))
    parts1, d10, d11 = edge(h1, asv1, adv1, c1)
    h2, asv2, adv2, c2 = _tc_mid(parts1, d10, d11, b1.reshape(1, H),
                                 batr, batc, gn1_w.reshape(1, H),
                                 gn1_b.reshape(1, H), gn1_ms.reshape(1, H),
                                 W2, as2.reshape(1, H), ad2.reshape(1, H))
    parts2, d20, d21 = edge(h2, asv2, adv2, c2)
    h3, asv3, adv3, c3 = _tc_mid(parts2, d20, d21, b2.reshape(1, H),
                                 batr, batc, gn2_w.reshape(1, H),
                                 gn2_b.reshape(1, H), gn2_ms.reshape(1, H),
                                 W3, as3.reshape(1, H), ad3.reshape(1, H))
    parts3, d30, d31 = edge(h3, asv3, adv3, c3)
    return _tc_fin(parts3, d30, d31, b3.reshape(1, H), batr,
                   hW1, hb1.reshape(1, H), hW2, hb2.reshape(1, A))


# speedup vs baseline: 37.4129x; 1.3867x over previous
import functools

import jax
import jax.numpy as jnp
from jax import lax
from jax.experimental import pallas as pl
from jax.experimental.pallas import tpu as pltpu
from jax.experimental.pallas import tpu_sc as plsc

N = 10000
E = 320000
H = 128
G = 16
A = 18

EC = 128
NCHUNK = E // EC
NW = 32
SB = 2
NITER = -(-NCHUNK // (NW * SB))
NP = 10112
NRC = NP // EC


def _pre_body(x_ref, w_ref, as_ref, ad_ref, h_ref, asv_ref, adv_ref, c_ref):
    h = jnp.dot(x_ref[...], w_ref[...], preferred_element_type=jnp.float32)
    h_ref[...] = h
    asv = jnp.sum(h * as_ref[...], axis=1, keepdims=True)
    adv = jnp.sum(h * ad_ref[...], axis=1, keepdims=True)
    asv_ref[...] = asv
    adv_ref[...] = adv
    m = jnp.max(asv) + jnp.max(adv)
    c_ref[...] = jnp.full((1, 1), 1.0, jnp.float32) * jnp.maximum(m, 0.2 * m)


def _tc_pre(x, w, a_s, a_d):
    return pl.pallas_call(
        _pre_body,
        out_shape=[
            jax.ShapeDtypeStruct((N, H), jnp.float32),
            jax.ShapeDtypeStruct((N, 1), jnp.float32),
            jax.ShapeDtypeStruct((N, 1), jnp.float32),
            jax.ShapeDtypeStruct((1, 1), jnp.float32),
        ],
    )(x, w, a_s, a_d)


def _combine(parts_ref, den0_ref, den1_ref, b_ref):
    hsum = parts_ref[0] + parts_ref[1]
    den = den0_ref[...] + den1_ref[...]
    return hsum / (den + 1e-16) + b_ref[...]


def _mid_body(parts_ref, den0_ref, den1_ref, b_ref, batr_ref, batc_ref,
              gw_ref, gb_ref, gms_ref, w_ref, as_ref, ad_ref,
              h_ref, asv_ref, adv_ref, c_ref):
    h = _combine(parts_ref, den0_ref, den1_ref, b_ref)
    gi = lax.broadcasted_iota(jnp.int32, (G, N), 0)
    oh = (gi == batr_ref[...]).astype(jnp.float32)
    gj = lax.broadcasted_iota(jnp.int32, (N, G), 1)
    ohT = (gj == batc_ref[...]).astype(jnp.float32)
    cnt = jnp.maximum(jnp.sum(oh, axis=1, keepdims=True), 1.0)
    mean = jnp.dot(oh, h, preferred_element_type=jnp.float32) / cnt
    xc = h - gms_ref[...] * jnp.dot(ohT, mean,
                                    preferred_element_type=jnp.float32)
    var = jnp.dot(oh, xc * xc, preferred_element_type=jnp.float32) / cnt
    hn = gw_ref[...] * xc / jnp.sqrt(
        jnp.dot(ohT, var, preferred_element_type=jnp.float32) + 1e-5
    ) + gb_ref[...]
    hn = jnp.maximum(hn, 0.01 * hn)
    h2 = jnp.dot(hn, w_ref[...], preferred_element_type=jnp.float32)
    h_ref[...] = h2
    asv = jnp.sum(h2 * as_ref[...], axis=1, keepdims=True)
    adv = jnp.sum(h2 * ad_ref[...], axis=1, keepdims=True)
    asv_ref[...] = asv
    adv_ref[...] = adv
    m = jnp.max(asv) + jnp.max(adv)
    c_ref[...] = jnp.full((1, 1), 1.0, jnp.float32) * jnp.maximum(m, 0.2 * m)


def _tc_mid(parts, den0, den1, b, batr, batc, gw, gb, gms, w, a_s, a_d):
    return pl.pallas_call(
        _mid_body,
        out_shape=[
            jax.ShapeDtypeStruct((N, H), jnp.float32),
            jax.ShapeDtypeStruct((N, 1), jnp.float32),
            jax.ShapeDtypeStruct((N, 1), jnp.float32),
            jax.ShapeDtypeStruct((1, 1), jnp.float32),
        ],
    )(parts, den0, den1, b, batr, batc, gw, gb, gms, w, a_s, a_d)


def _fin_body(parts_ref, den0_ref, den1_ref, b_ref, batr_ref,
              hw1_ref, hb1_ref, hw2_ref, hb2_ref, out_ref):
    h = _combine(parts_ref, den0_ref, den1_ref, b_ref)
    gi = lax.broadcasted_iota(jnp.int32, (G, N), 0)
    oh = (gi == batr_ref[...]).astype(jnp.float32)
    pooled = jnp.dot(oh, h, preferred_element_type=jnp.float32)
    z = jnp.dot(pooled, hw1_ref[...],
                preferred_element_type=jnp.float32) + hb1_ref[...]
    z = jnp.maximum(z, 0.01 * z)
    out_ref[...] = jnp.dot(z, hw2_ref[...],
                           preferred_element_type=jnp.float32) + hb2_ref[...]


def _tc_fin(parts, den0, den1, b, batr, hw1, hb1, hw2, hb2):
    return pl.pallas_call(
        _fin_body,
        out_shape=jax.ShapeDtypeStruct((G, A), jnp.float32),
    )(parts, den0, den1, b, batr, hw1, hb1, hw2, hb2)


_MESH = plsc.VectorSubcoreMesh(core_axis_name="c", subcore_axis_name="s")


@functools.partial(
    pl.kernel,
    out_type=[
        jax.ShapeDtypeStruct((2, NP, H), jnp.float32),
        jax.ShapeDtypeStruct((2, NP), jnp.float32),
    ],
    mesh=_MESH,
    scratch_types=[
        pltpu.VMEM_SHARED((NP, H), jnp.float32),
        pltpu.VMEM_SHARED((NP,), jnp.float32),
        pltpu.VMEM_SHARED((N,), jnp.float32),
        pltpu.VMEM_SHARED((N,), jnp.float32),
        [pltpu.VMEM((EC,), jnp.int32)] * SB,
        [pltpu.VMEM((EC,), jnp.int32)] * SB,
        [pltpu.VMEM((EC,), jnp.float32)] * SB,
        [pltpu.VMEM((EC,), jnp.float32)] * SB,
        [pltpu.VMEM((EC,), jnp.float32)] * SB,
        [pltpu.VMEM((EC, H), jnp.float32)] * SB,
        pltpu.VMEM((EC,), jnp.float32),
        [pltpu.SemaphoreType.DMA] * SB,
        [pltpu.SemaphoreType.DMA] * SB,
        [pltpu.SemaphoreType.DMA] * SB,
    ],
)
def _sc_edge(h_hbm, asv_hbm, adv_hbm, src_hbm, dst_hbm, c_hbm,
             out_hbm, den_hbm,
             out_sh, den_sh, asv_sh, adv_sh,
             srcv, dstv, asg, adg, pv, rows, cv,
             sem_r, sem_a, sem_b):
    c = lax.axis_index("c")
    s = lax.axis_index("s")
    wid = c * 16 + s

    zero16 = jnp.zeros((16,), jnp.float32)
    zrows = rows[0]

    def zbody(i, carry):
        for k in range(H // 16):
            zrows[i, pl.ds(k * 16, 16)] = zero16
        return carry

    lax.fori_loop(0, EC, zbody, 0)

    @pl.when(s == 0)
    def _():
        pltpu.sync_copy(asv_hbm, asv_sh)
        pltpu.sync_copy(adv_hbm, adv_sh)

    pltpu.sync_copy(c_hbm, cv)

    for t in range(5):
        idxr = s + 16 * t

        @pl.when(idxr < NRC)
        def _():
            pltpu.sync_copy(zrows, out_sh.at[pl.ds(idxr * EC, EC)])
            pltpu.sync_copy(zrows.at[0], den_sh.at[pl.ds(idxr * EC, EC)])

    plsc.subcore_barrier()
    cvec = cv[pl.ds(0, 16)]

    def chunk_id(t, q):
        return wid + NW * (SB * t + q)

    def iter_body(t, carry):
        for q in range(SB):
            cid = chunk_id(t, q)

            @pl.when(cid < NCHUNK)
            def _():
                base = cid * EC
                pltpu.sync_copy(src_hbm.at[pl.ds(base, EC)], srcv[q])
                pltpu.sync_copy(dst_hbm.at[pl.ds(base, EC)], dstv[q])
                pltpu.async_copy(h_hbm.at[srcv[q]], rows[q], sem_r[q])
                pltpu.async_copy(asv_sh.at[srcv[q]], asg[q], sem_a[q])
                pltpu.async_copy(adv_sh.at[dstv[q]], adg[q], sem_b[q])

        for q in range(SB):
            cid = chunk_id(t, q)

            @pl.when(cid < NCHUNK)
            def _():
                pltpu.make_async_copy(h_hbm.at[srcv[q]],
                                      rows[q], sem_r[q]).wait()
                pltpu.make_async_copy(asv_sh.at[srcv[q]],
                                      asg[q], sem_a[q]).wait()
                pltpu.make_async_copy(adv_sh.at[dstv[q]],
                                      adg[q], sem_b[q]).wait()
                for j in range(EC // 16):
                    sl = pl.ds(j * 16, 16)
                    e = asg[q][sl] + adg[q][sl]
                    e = jnp.maximum(e, 0.2 * e)
                    pv[q][sl] = jnp.exp(e - cvec)

                def rbody(jj, carry2):
                    pvec = pv[q][pl.ds(jj * 16, 16)]
                    for m in range(16):
                        psc = pvec[m]
                        i = jj * 16 + m
                        for k in range(H // 16):
                            slk = pl.ds(k * 16, 16)
                            rows[q][i, slk] = rows[q][i, slk] * psc
                    return carry2

                lax.fori_loop(0, EC // 16, rbody, 0)
                pltpu.sync_copy(pv[q], den_sh.at[dstv[q]], add=True)
                pltpu.sync_copy(rows[q], out_sh.at[dstv[q]], add=True)

        return carry

    lax.fori_loop(0, NITER, iter_body, 0)
    plsc.subcore_barrier()

    for t in range(5):
        idxr = s + 16 * t

        @pl.when(idxr < NRC)
        def _():
            pltpu.sync_copy(out_sh.at[pl.ds(idxr * EC, EC)],
                            out_hbm.at[c, pl.ds(idxr * EC, EC)])
            pltpu.sync_copy(den_sh.at[pl.ds(idxr * EC, EC)],
                            den_hbm.at[c, pl.ds(idxr * EC, EC)])


def kernel(x, edge_index, batch, W1, as1, ad1, b1, W2, as2, ad2, b2,
           W3, as3, ad3, b3, gn1_w, gn1_b, gn1_ms, gn2_w, gn2_b, gn2_ms,
           hW1, hb1, hW2, hb2):
    src = edge_index[0]
    dst = edge_index[1]
    batr = batch.reshape(1, N)
    batc = batch.reshape(N, 1)

    def edge(h, asv, adv, c1):
        c128 = jnp.broadcast_to(c1.reshape(()), (EC,))
        parts, den = _sc_edge(h, asv.reshape(N), adv.reshape(N),
                              src, dst, c128)
        return (parts[:, :N, :], den[0, :N].reshape(N, 1),
                den[1, :N].reshape(N, 1))

    h1, asv1, adv1, c1 = _tc_pre(x, W1, as1.reshape(1, H), ad1.reshape(1, H))
    parts1, d10, d11 = edge(h1, asv1, adv1, c1)
    h2, asv2, adv2, c2 = _tc_mid(parts1, d10, d11, b1.reshape(1, H),
                                 batr, batc, gn1_w.reshape(1, H),
                                 gn1_b.reshape(1, H), gn1_ms.reshape(1, H),
                                 W2, as2.reshape(1, H), ad2.reshape(1, H))
    parts2, d20, d21 = edge(h2, asv2, adv2, c2)
    h3, asv3, adv3, c3 = _tc_mid(parts2, d20, d21, b2.reshape(1, H),
                                 batr, batc, gn2_w.reshape(1, H),
                                 gn2_b.reshape(1, H), gn2_ms.reshape(1, H),
                                 W3, as3.reshape(1, H), ad3.reshape(1, H))
    parts3, d30, d31 = edge(h3, asv3, adv3, c3)
    return _tc_fin(parts3, d30, d31, b3.reshape(1, H), batr,
                   hW1, hb1.reshape(1, H), hW2, hb2.reshape(1, A))


# speedup vs baseline: 40.3098x; 1.0774x over previous
import functools

import jax
import jax.numpy as jnp
from jax import lax
from jax.experimental import pallas as pl
from jax.experimental.pallas import tpu as pltpu
from jax.experimental.pallas import tpu_sc as plsc

N = 10000
E = 320000
H = 128
G = 16
A = 18

EC = 128
NCHUNK = E // EC
NW = 32
SB = 2
NITER = -(-NCHUNK // (NW * SB))
NP = 10112
NRC = NP // EC


def _pre_body(x_ref, w_ref, as_ref, ad_ref, h_ref, asv_ref, adv_ref, c_ref):
    h = jnp.dot(x_ref[...], w_ref[...], preferred_element_type=jnp.float32)
    h_ref[...] = h
    asv = jnp.sum(h * as_ref[...], axis=1, keepdims=True)
    adv = jnp.sum(h * ad_ref[...], axis=1, keepdims=True)
    asv_ref[...] = asv
    adv_ref[...] = adv
    m = jnp.max(asv) + jnp.max(adv)
    c_ref[...] = jnp.full((1, 1), 1.0, jnp.float32) * jnp.maximum(m, 0.2 * m)


def _tc_pre(x, w, a_s, a_d):
    return pl.pallas_call(
        _pre_body,
        out_shape=[
            jax.ShapeDtypeStruct((N, H), jnp.float32),
            jax.ShapeDtypeStruct((N, 1), jnp.float32),
            jax.ShapeDtypeStruct((N, 1), jnp.float32),
            jax.ShapeDtypeStruct((1, 1), jnp.float32),
        ],
    )(x, w, a_s, a_d)


def _combine(parts_ref, den0_ref, den1_ref, b_ref):
    hsum = parts_ref[0] + parts_ref[1]
    den = den0_ref[...] + den1_ref[...]
    return hsum / (den + 1e-16) + b_ref[...]


def _mid_body(parts_ref, den0_ref, den1_ref, b_ref, batr_ref, batc_ref,
              gw_ref, gb_ref, gms_ref, w_ref, as_ref, ad_ref,
              h_ref, asv_ref, adv_ref, c_ref):
    h = _combine(parts_ref, den0_ref, den1_ref, b_ref)
    gi = lax.broadcasted_iota(jnp.int32, (G, N), 0)
    oh = (gi == batr_ref[...]).astype(jnp.float32)
    gj = lax.broadcasted_iota(jnp.int32, (N, G), 1)
    ohT = (gj == batc_ref[...]).astype(jnp.float32)
    cnt = jnp.maximum(jnp.sum(oh, axis=1, keepdims=True), 1.0)
    mean = jnp.dot(oh, h, preferred_element_type=jnp.float32) / cnt
    xc = h - gms_ref[...] * jnp.dot(ohT, mean,
                                    preferred_element_type=jnp.float32)
    var = jnp.dot(oh, xc * xc, preferred_element_type=jnp.float32) / cnt
    hn = gw_ref[...] * xc / jnp.sqrt(
        jnp.dot(ohT, var, preferred_element_type=jnp.float32) + 1e-5
    ) + gb_ref[...]
    hn = jnp.maximum(hn, 0.01 * hn)
    h2 = jnp.dot(hn, w_ref[...], preferred_element_type=jnp.float32)
    h_ref[...] = h2
    asv = jnp.sum(h2 * as_ref[...], axis=1, keepdims=True)
    adv = jnp.sum(h2 * ad_ref[...], axis=1, keepdims=True)
    asv_ref[...] = asv
    adv_ref[...] = adv
    m = jnp.max(asv) + jnp.max(adv)
    c_ref[...] = jnp.full((1, 1), 1.0, jnp.float32) * jnp.maximum(m, 0.2 * m)


def _tc_mid(parts, den0, den1, b, batr, batc, gw, gb, gms, w, a_s, a_d):
    return pl.pallas_call(
        _mid_body,
        out_shape=[
            jax.ShapeDtypeStruct((N, H), jnp.float32),
            jax.ShapeDtypeStruct((N, 1), jnp.float32),
            jax.ShapeDtypeStruct((N, 1), jnp.float32),
            jax.ShapeDtypeStruct((1, 1), jnp.float32),
        ],
    )(parts, den0, den1, b, batr, batc, gw, gb, gms, w, a_s, a_d)


def _fin_body(parts_ref, den0_ref, den1_ref, b_ref, batr_ref,
              hw1_ref, hb1_ref, hw2_ref, hb2_ref, out_ref):
    h = _combine(parts_ref, den0_ref, den1_ref, b_ref)
    gi = lax.broadcasted_iota(jnp.int32, (G, N), 0)
    oh = (gi == batr_ref[...]).astype(jnp.float32)
    pooled = jnp.dot(oh, h, preferred_element_type=jnp.float32)
    z = jnp.dot(pooled, hw1_ref[...],
                preferred_element_type=jnp.float32) + hb1_ref[...]
    z = jnp.maximum(z, 0.01 * z)
    out_ref[...] = jnp.dot(z, hw2_ref[...],
                           preferred_element_type=jnp.float32) + hb2_ref[...]


def _tc_fin(parts, den0, den1, b, batr, hw1, hb1, hw2, hb2):
    return pl.pallas_call(
        _fin_body,
        out_shape=jax.ShapeDtypeStruct((G, A), jnp.float32),
    )(parts, den0, den1, b, batr, hw1, hb1, hw2, hb2)


_MESH = plsc.VectorSubcoreMesh(core_axis_name="c", subcore_axis_name="s")


@functools.partial(
    pl.kernel,
    out_type=[
        jax.ShapeDtypeStruct((2, NP, H), jnp.float32),
        jax.ShapeDtypeStruct((2, NP), jnp.float32),
    ],
    mesh=_MESH,
    scratch_types=[
        pltpu.VMEM_SHARED((NP, H), jnp.float32),
        pltpu.VMEM_SHARED((NP,), jnp.float32),
        pltpu.VMEM_SHARED((N,), jnp.float32),
        pltpu.VMEM_SHARED((N,), jnp.float32),
        [pltpu.VMEM((EC,), jnp.int32)] * SB,
        [pltpu.VMEM((EC,), jnp.int32)] * SB,
        [pltpu.VMEM((EC,), jnp.float32)] * SB,
        [pltpu.VMEM((EC,), jnp.float32)] * SB,
        [pltpu.VMEM((EC,), jnp.float32)] * SB,
        [pltpu.VMEM((EC, H), jnp.float32)] * SB,
        pltpu.VMEM((EC,), jnp.float32),
        [pltpu.SemaphoreType.DMA] * SB,
        [pltpu.SemaphoreType.DMA] * SB,
        [pltpu.SemaphoreType.DMA] * SB,
        [pltpu.SemaphoreType.DMA] * SB,
        [pltpu.SemaphoreType.DMA] * SB,
    ],
)
def _sc_edge(h_hbm, asv_hbm, adv_hbm, src_hbm, dst_hbm, c_hbm,
             out_hbm, den_hbm,
             out_sh, den_sh, asv_sh, adv_sh,
             srcv, dstv, asg, adg, pv, rows, cv,
             sem_r, sem_a, sem_b, sem_w1, sem_w2):
    c = lax.axis_index("c")
    s = lax.axis_index("s")
    wid = c * 16 + s

    zero16 = jnp.zeros((16,), jnp.float32)
    zrows = rows[0]

    def zbody(i, carry):
        for k in range(H // 16):
            zrows[i, pl.ds(k * 16, 16)] = zero16
        return carry

    lax.fori_loop(0, EC, zbody, 0)

    @pl.when(s == 0)
    def _():
        pltpu.sync_copy(asv_hbm, asv_sh)
        pltpu.sync_copy(adv_hbm, adv_sh)

    pltpu.sync_copy(c_hbm, cv)

    for t in range(5):
        idxr = s + 16 * t

        @pl.when(idxr < NRC)
        def _():
            pltpu.sync_copy(zrows, out_sh.at[pl.ds(idxr * EC, EC)])
            pltpu.sync_copy(zrows.at[0], den_sh.at[pl.ds(idxr * EC, EC)])

    plsc.subcore_barrier()
    cvec = cv[pl.ds(0, 16)]

    def cid_of(k):
        return wid + NW * k

    def issue(k, b):
        base = cid_of(k) * EC
        pltpu.sync_copy(src_hbm.at[pl.ds(base, EC)], srcv[b])
        pltpu.sync_copy(dst_hbm.at[pl.ds(base, EC)], dstv[b])
        pltpu.async_copy(h_hbm.at[srcv[b]], rows[b], sem_r[b])
        pltpu.async_copy(asv_sh.at[srcv[b]], asg[b], sem_a[b])
        pltpu.async_copy(adv_sh.at[dstv[b]], adg[b], sem_b[b])

    def consume(b):
        pltpu.make_async_copy(h_hbm.at[srcv[b]], rows[b], sem_r[b]).wait()
        pltpu.make_async_copy(asv_sh.at[srcv[b]], asg[b], sem_a[b]).wait()
        pltpu.make_async_copy(adv_sh.at[dstv[b]], adg[b], sem_b[b]).wait()
        for j in range(EC // 16):
            sl = pl.ds(j * 16, 16)
            e = asg[b][sl] + adg[b][sl]
            e = jnp.maximum(e, 0.2 * e)
            pv[b][sl] = jnp.exp(e - cvec)

        def rbody(jj, carry2):
            pvec = pv[b][pl.ds(jj * 16, 16)]
            for m in range(16):
                psc = pvec[m]
                i = jj * 16 + m
                for k in range(H // 16):
                    slk = pl.ds(k * 16, 16)
                    rows[b][i, slk] = rows[b][i, slk] * psc
            return carry2

        lax.fori_loop(0, EC // 16, rbody, 0)
        pltpu.async_copy(rows[b], out_sh.at[dstv[b]], sem_w1[b], add=True)
        pltpu.async_copy(pv[b], den_sh.at[dstv[b]], sem_w2[b], add=True)

    def wait_scatter(b):
        pltpu.make_async_copy(rows[b], out_sh.at[dstv[b]], sem_w1[b]).wait()
        pltpu.make_async_copy(pv[b], den_sh.at[dstv[b]], sem_w2[b]).wait()

    for b in range(SB):
        @pl.when(cid_of(b) < NCHUNK)
        def _():
            issue(b, b)

    def iter_body(t, carry):
        for b in range(SB):
            k = SB * t + b
            kn = k + SB

            @pl.when(cid_of(k) < NCHUNK)
            def _():
                consume(b)

            @pl.when(cid_of(kn) < NCHUNK)
            def _():
                wait_scatter(b)
                issue(kn, b)

        return carry

    lax.fori_loop(0, NITER, iter_body, 0)

    m_chunks = (NCHUNK - wid + NW - 1) // NW
    for b in range(SB):
        kb = jnp.where((m_chunks - 1) % SB == b, m_chunks - 1, m_chunks - 2)

        @pl.when(kb >= 0)
        def _():
            wait_scatter(b)

    plsc.subcore_barrier()

    for t in range(5):
        idxr = s + 16 * t

        @pl.when(idxr < NRC)
        def _():
            pltpu.sync_copy(out_sh.at[pl.ds(idxr * EC, EC)],
                            out_hbm.at[c, pl.ds(idxr * EC, EC)])
            pltpu.sync_copy(den_sh.at[pl.ds(idxr * EC, EC)],
                            den_hbm.at[c, pl.ds(idxr * EC, EC)])


def kernel(x, edge_index, batch, W1, as1, ad1, b1, W2, as2, ad2, b2,
           W3, as3, ad3, b3, gn1_w, gn1_b, gn1_ms, gn2_w, gn2_b, gn2_ms,
           hW1, hb1, hW2, hb2):
    src = edge_index[0]
    dst = edge_index[1]
    batr = batch.reshape(1, N)
    batc = batch.reshape(N, 1)

    def edge(h, asv, adv, c1):
        c128 = jnp.broadcast_to(c1.reshape(()), (EC,))
        parts, den = _sc_edge(h, asv.reshape(N), adv.reshape(N),
                              src, dst, c128)
        return (parts[:, :N, :], den[0, :N].reshape(N, 1),
                den[1, :N].reshape(N, 1))

    h1, asv1, adv1, c1 = _tc_pre(x, W1, as1.reshape(1, H), ad1.reshape(1, H))
    parts1, d10, d11 = edge(h1, asv1, adv1, c1)
    h2, asv2, adv2, c2 = _tc_mid(parts1, d10, d11, b1.reshape(1, H),
                                 batr, batc, gn1_w.reshape(1, H),
                                 gn1_b.reshape(1, H), gn1_ms.reshape(1, H),
                                 W2, as2.reshape(1, H), ad2.reshape(1, H))
    parts2, d20, d21 = edge(h2, asv2, adv2, c2)
    h3, asv3, adv3, c3 = _tc_mid(parts2, d20, d21, b2.reshape(1, H),
                                 batr, batc, gn2_w.reshape(1, H),
                                 gn2_b.reshape(1, H), gn2_ms.reshape(1, H),
                                 W3, as3.reshape(1, H), ad3.reshape(1, H))
    parts3, d30, d31 = edge(h3, asv3, adv3, c3)
    return _tc_fin(parts3, d30, d31, b3.reshape(1, H), batr,
                   hW1, hb1.reshape(1, H), hW2, hb2.reshape(1, A))


# speedup vs baseline: 47.3650x; 1.1750x over previous
import functools

import jax
import jax.numpy as jnp
from jax import lax
from jax.experimental import pallas as pl
from jax.experimental.pallas import tpu as pltpu
from jax.experimental.pallas import tpu_sc as plsc

N = 10000
E = 320000
H = 128
G = 16
A = 18

EC = 128
NCHUNK = E // EC
NW = 32
SB = 2
NITER = -(-NCHUNK // (NW * SB))
NP = 10112
NRC = NP // EC


def _pre_body(x_ref, w_ref, as_ref, ad_ref, h_ref, asv_ref, adv_ref, c_ref):
    h = jnp.dot(x_ref[...], w_ref[...], preferred_element_type=jnp.float32)
    h_ref[...] = h
    asv = jnp.sum(h * as_ref[...], axis=1, keepdims=True)
    adv = jnp.sum(h * ad_ref[...], axis=1, keepdims=True)
    asv_ref[...] = asv
    adv_ref[...] = adv
    m = jnp.max(asv) + jnp.max(adv)
    c_ref[...] = jnp.full((1, 1), 1.0, jnp.float32) * jnp.maximum(m, 0.2 * m)


def _tc_pre(x, w, a_s, a_d):
    return pl.pallas_call(
        _pre_body,
        out_shape=[
            jax.ShapeDtypeStruct((N, H), jnp.float32),
            jax.ShapeDtypeStruct((N, 1), jnp.float32),
            jax.ShapeDtypeStruct((N, 1), jnp.float32),
            jax.ShapeDtypeStruct((1, 1), jnp.float32),
        ],
    )(x, w, a_s, a_d)


def _combine(parts_ref, den0_ref, den1_ref, b_ref):
    hsum = parts_ref[0] + parts_ref[1]
    den = den0_ref[...] + den1_ref[...]
    return hsum / (den + 1e-16) + b_ref[...]


def _mid_body(parts_ref, den0_ref, den1_ref, b_ref, batr_ref, batc_ref,
              gw_ref, gb_ref, gms_ref, w_ref, as_ref, ad_ref,
              h_ref, asv_ref, adv_ref, c_ref):
    h = _combine(parts_ref, den0_ref, den1_ref, b_ref)
    gi = lax.broadcasted_iota(jnp.int32, (G, N), 0)
    oh = (gi == batr_ref[...]).astype(jnp.float32)
    gj = lax.broadcasted_iota(jnp.int32, (N, G), 1)
    ohT = (gj == batc_ref[...]).astype(jnp.float32)
    cnt = jnp.maximum(jnp.sum(oh, axis=1, keepdims=True), 1.0)
    mean = jnp.dot(oh, h, preferred_element_type=jnp.float32) / cnt
    xc = h - gms_ref[...] * jnp.dot(ohT, mean,
                                    preferred_element_type=jnp.float32)
    var = jnp.dot(oh, xc * xc, preferred_element_type=jnp.float32) / cnt
    hn = gw_ref[...] * xc / jnp.sqrt(
        jnp.dot(ohT, var, preferred_element_type=jnp.float32) + 1e-5
    ) + gb_ref[...]
    hn = jnp.maximum(hn, 0.01 * hn)
    h2 = jnp.dot(hn, w_ref[...], preferred_element_type=jnp.float32)
    h_ref[...] = h2
    asv = jnp.sum(h2 * as_ref[...], axis=1, keepdims=True)
    adv = jnp.sum(h2 * ad_ref[...], axis=1, keepdims=True)
    asv_ref[...] = asv
    adv_ref[...] = adv
    m = jnp.max(asv) + jnp.max(adv)
    c_ref[...] = jnp.full((1, 1), 1.0, jnp.float32) * jnp.maximum(m, 0.2 * m)


def _tc_mid(parts, den0, den1, b, batr, batc, gw, gb, gms, w, a_s, a_d):
    return pl.pallas_call(
        _mid_body,
        out_shape=[
            jax.ShapeDtypeStruct((N, H), jnp.float32),
            jax.ShapeDtypeStruct((N, 1), jnp.float32),
            jax.ShapeDtypeStruct((N, 1), jnp.float32),
            jax.ShapeDtypeStruct((1, 1), jnp.float32),
        ],
    )(parts, den0, den1, b, batr, batc, gw, gb, gms, w, a_s, a_d)


def _fin_body(parts_ref, den0_ref, den1_ref, b_ref, batr_ref,
              hw1_ref, hb1_ref, hw2_ref, hb2_ref, out_ref):
    h = _combine(parts_ref, den0_ref, den1_ref, b_ref)
    gi = lax.broadcasted_iota(jnp.int32, (G, N), 0)
    oh = (gi == batr_ref[...]).astype(jnp.float32)
    pooled = jnp.dot(oh, h, preferred_element_type=jnp.float32)
    z = jnp.dot(pooled, hw1_ref[...],
                preferred_element_type=jnp.float32) + hb1_ref[...]
    z = jnp.maximum(z, 0.01 * z)
    out_ref[...] = jnp.dot(z, hw2_ref[...],
                           preferred_element_type=jnp.float32) + hb2_ref[...]


def _tc_fin(parts, den0, den1, b, batr, hw1, hb1, hw2, hb2):
    return pl.pallas_call(
        _fin_body,
        out_shape=jax.ShapeDtypeStruct((G, A), jnp.float32),
    )(parts, den0, den1, b, batr, hw1, hb1, hw2, hb2)


_MESH = plsc.VectorSubcoreMesh(core_axis_name="c", subcore_axis_name="s")


@functools.partial(
    pl.kernel,
    out_type=[
        jax.ShapeDtypeStruct((2, NP, H), jnp.float32),
        jax.ShapeDtypeStruct((2, NP), jnp.float32),
    ],
    mesh=_MESH,
    scratch_types=[
        pltpu.VMEM_SHARED((NP, H), jnp.float32),
        pltpu.VMEM_SHARED((NP,), jnp.float32),
        pltpu.VMEM_SHARED((N,), jnp.float32),
        pltpu.VMEM_SHARED((N,), jnp.float32),
        [pltpu.VMEM((EC,), jnp.int32)] * SB,
        [pltpu.VMEM((2, EC // 2), jnp.int32)] * SB,
        [pltpu.VMEM((EC,), jnp.float32)] * SB,
        [pltpu.VMEM((EC,), jnp.float32)] * SB,
        [pltpu.VMEM((2, EC // 2), jnp.float32)] * SB,
        [pltpu.VMEM((EC, H), jnp.float32)] * SB,
        pltpu.VMEM((EC,), jnp.float32),
        [pltpu.SemaphoreType.DMA] * SB,
        [pltpu.SemaphoreType.DMA] * SB,
        [pltpu.SemaphoreType.DMA] * SB,
        [[pltpu.SemaphoreType.DMA] * 2] * SB,
        [[pltpu.SemaphoreType.DMA] * 2] * SB,
    ],
)
def _sc_edge(h_hbm, asv_hbm, adv_hbm, src_hbm, dst2_hbm, c_hbm,
             out_hbm, den_hbm,
             out_sh, den_sh, asv_sh, adv_sh,
             srcv, dstv, asg, adg, pv, rows, cv,
             sem_r, sem_a, sem_b, sem_w1, sem_w2):
    c = lax.axis_index("c")
    s = lax.axis_index("s")
    wid = c * 16 + s

    zero16 = jnp.zeros((16,), jnp.float32)
    zrows = rows[0]

    def zbody(i, carry):
        for k in range(H // 16):
            zrows[i, pl.ds(k * 16, 16)] = zero16
        return carry

    lax.fori_loop(0, EC, zbody, 0)

    @pl.when(s == 0)
    def _():
        pltpu.sync_copy(asv_hbm, asv_sh)
        pltpu.sync_copy(adv_hbm, adv_sh)

    pltpu.sync_copy(c_hbm, cv)

    for t in range(5):
        idxr = s + 16 * t

        @pl.when(idxr < NRC)
        def _():
            pltpu.sync_copy(zrows, out_sh.at[pl.ds(idxr * EC, EC)])
            pltpu.sync_copy(zrows.at[0], den_sh.at[pl.ds(idxr * EC, EC)])

    plsc.subcore_barrier()
    cvec = cv[pl.ds(0, 16)]

    def cid_of(k):
        return wid + NW * k

    HEC = EC // 2

    def issue(k, b):
        base = cid_of(k) * EC
        pltpu.sync_copy(src_hbm.at[pl.ds(base, EC)], srcv[b])
        pltpu.async_copy(h_hbm.at[srcv[b]], rows[b], sem_r[b])
        pltpu.async_copy(asv_sh.at[srcv[b]], asg[b], sem_a[b])
        pltpu.sync_copy(dst2_hbm.at[pl.ds(cid_of(k) * 2, 2)], dstv[b])
        for j in range(2):
            pltpu.async_copy(adv_sh.at[dstv[b].at[j]],
                             adg[b].at[pl.ds(j * HEC, HEC)], sem_b[b])

    def consume(b):
        pltpu.make_async_copy(h_hbm.at[srcv[b]], rows[b], sem_r[b]).wait()
        pltpu.make_async_copy(asv_sh.at[srcv[b]], asg[b], sem_a[b]).wait()
        for j in range(2):
            pltpu.make_async_copy(adv_sh.at[dstv[b].at[j]],
                                  adg[b].at[pl.ds(j * HEC, HEC)],
                                  sem_b[b]).wait()
        for j2 in range(2):
            for jj in range(HEC // 16):
                sle = pl.ds(j2 * HEC + jj * 16, 16)
                e = asg[b][sle] + adg[b][sle]
                e = jnp.maximum(e, 0.2 * e)
                pv[b][j2, pl.ds(jj * 16, 16)] = jnp.exp(e - cvec)

        for j2 in range(2):
            def rbody(jj, carry2, j2=j2):
                pvec = pv[b][j2, pl.ds(jj * 16, 16)]
                for m in range(16):
                    psc = pvec[m]
                    i = j2 * HEC + jj * 16 + m
                    for k in range(H // 16):
                        slk = pl.ds(k * 16, 16)
                        rows[b][i, slk] = rows[b][i, slk] * psc
                return carry2

            lax.fori_loop(0, HEC // 16, rbody, 0)
            pltpu.async_copy(rows[b].at[pl.ds(j2 * HEC, HEC)],
                             out_sh.at[dstv[b].at[j2]],
                             sem_w1[b][j2], add=True)
            pltpu.async_copy(pv[b].at[j2], den_sh.at[dstv[b].at[j2]],
                             sem_w2[b][j2], add=True)

    def wait_scatter(b):
        for j2 in range(2):
            pltpu.make_async_copy(rows[b].at[pl.ds(j2 * HEC, HEC)],
                                  out_sh.at[dstv[b].at[j2]],
                                  sem_w1[b][j2]).wait()
            pltpu.make_async_copy(pv[b].at[j2], den_sh.at[dstv[b].at[j2]],
                                  sem_w2[b][j2]).wait()

    def issue_steady(k, b):
        base = cid_of(k) * EC
        pltpu.sync_copy(src_hbm.at[pl.ds(base, EC)], srcv[b])
        pltpu.async_copy(asv_sh.at[srcv[b]], asg[b], sem_a[b])
        wait_scatter(b)
        pltpu.sync_copy(dst2_hbm.at[pl.ds(cid_of(k) * 2, 2)], dstv[b])
        pltpu.async_copy(h_hbm.at[srcv[b]], rows[b], sem_r[b])
        for j in range(2):
            pltpu.async_copy(adv_sh.at[dstv[b].at[j]],
                             adg[b].at[pl.ds(j * HEC, HEC)], sem_b[b])

    for b in range(SB):
        @pl.when(cid_of(b) < NCHUNK)
        def _():
            issue(b, b)

    def iter_body(t, carry):
        for b in range(SB):
            k = SB * t + b
            kn = k + SB

            @pl.when(cid_of(k) < NCHUNK)
            def _():
                consume(b)

            @pl.when(cid_of(kn) < NCHUNK)
            def _():
                issue_steady(kn, b)

        return carry

    lax.fori_loop(0, NITER, iter_body, 0)

    m_chunks = (NCHUNK - wid + NW - 1) // NW
    for b in range(SB):
        kb = jnp.where((m_chunks - 1) % SB == b, m_chunks - 1, m_chunks - 2)

        @pl.when(kb >= 0)
        def _():
            wait_scatter(b)

    plsc.subcore_barrier()

    for t in range(5):
        idxr = s + 16 * t

        @pl.when(idxr < NRC)
        def _():
            pltpu.sync_copy(out_sh.at[pl.ds(idxr * EC, EC)],
                            out_hbm.at[c, pl.ds(idxr * EC, EC)])
            pltpu.sync_copy(den_sh.at[pl.ds(idxr * EC, EC)],
                            den_hbm.at[c, pl.ds(idxr * EC, EC)])


def kernel(x, edge_index, batch, W1, as1, ad1, b1, W2, as2, ad2, b2,
           W3, as3, ad3, b3, gn1_w, gn1_b, gn1_ms, gn2_w, gn2_b, gn2_ms,
           hW1, hb1, hW2, hb2):
    src = edge_index[0]
    dst2 = edge_index[1].reshape(E // (EC // 2), EC // 2)
    batr = batch.reshape(1, N)
    batc = batch.reshape(N, 1)

    def edge(h, asv, adv, c1):
        c128 = jnp.broadcast_to(c1.reshape(()), (EC,))
        parts, den = _sc_edge(h, asv.reshape(N), adv.reshape(N),
                              src, dst2, c128)
        return (parts[:, :N, :], den[0, :N].reshape(N, 1),
                den[1, :N].reshape(N, 1))

    h1, asv1, adv1, c1 = _tc_pre(x, W1, as1.reshape(1, H), ad1.reshape(1, H))
    parts1, d10, d11 = edge(h1, asv1, adv1, c1)
    h2, asv2, adv2, c2 = _tc_mid(parts1, d10, d11, b1.reshape(1, H),
                                 batr, batc, gn1_w.reshape(1, H),
                                 gn1_b.reshape(1, H), gn1_ms.reshape(1, H),
                                 W2, as2.reshape(1, H), ad2.reshape(1, H))
    parts2, d20, d21 = edge(h2, asv2, adv2, c2)
    h3, asv3, adv3, c3 = _tc_mid(parts2, d20, d21, b2.reshape(1, H),
                                 batr, batc, gn2_w.reshape(1, H),
                                 gn2_b.reshape(1, H), gn2_ms.reshape(1, H),
                                 W3, as3.reshape(1, H), ad3.reshape(1, H))
    parts3, d30, d31 = edge(h3, asv3, adv3, c3)
    return _tc_fin(parts3, d30, d31, b3.reshape(1, H), batr,
                   hW1, hb1.reshape(1, H), hW2, hb2.reshape(1, A))


# speedup vs baseline: 56.4270x; 1.1913x over previous
import functools

import jax
import jax.numpy as jnp
from jax import lax
from jax.experimental import pallas as pl
from jax.experimental.pallas import tpu as pltpu
from jax.experimental.pallas import tpu_sc as plsc

N = 10000
E = 320000
H = 128
G = 16
A = 18

EC = 128
NCHUNK = E // EC
NW = 32
SB = 2
NITER = -(-NCHUNK // (NW * SB))
NP = 10112
NRC = NP // EC


def _pre_body(x_ref, w_ref, as_ref, ad_ref, h_ref, asv_ref, adv_ref, c_ref):
    h = jnp.dot(x_ref[...], w_ref[...], preferred_element_type=jnp.float32)
    h_ref[...] = h
    asv = jnp.sum(h * as_ref[...], axis=1, keepdims=True)
    adv = jnp.sum(h * ad_ref[...], axis=1, keepdims=True)
    asv_ref[...] = asv
    adv_ref[...] = adv
    m = jnp.max(asv) + jnp.max(adv)
    c_ref[...] = jnp.full((1, 1), 1.0, jnp.float32) * jnp.maximum(m, 0.2 * m)


def _tc_pre(x, w, a_s, a_d):
    return pl.pallas_call(
        _pre_body,
        out_shape=[
            jax.ShapeDtypeStruct((N, H), jnp.float32),
            jax.ShapeDtypeStruct((N, 1), jnp.float32),
            jax.ShapeDtypeStruct((N, 1), jnp.float32),
            jax.ShapeDtypeStruct((1, 1), jnp.float32),
        ],
    )(x, w, a_s, a_d)


def _combine(parts_ref, den0_ref, den1_ref, b_ref):
    hsum = parts_ref[0] + parts_ref[1]
    den = den0_ref[...] + den1_ref[...]
    return hsum / (den + 1e-16) + b_ref[...]


def _mid_body(parts_ref, den0_ref, den1_ref, b_ref, batr_ref, batc_ref,
              gw_ref, gb_ref, gms_ref, w_ref, as_ref, ad_ref,
              h_ref, asv_ref, adv_ref, c_ref):
    h = _combine(parts_ref, den0_ref, den1_ref, b_ref)
    gi = lax.broadcasted_iota(jnp.int32, (G, N), 0)
    oh = (gi == batr_ref[...]).astype(jnp.float32)
    gj = lax.broadcasted_iota(jnp.int32, (N, G), 1)
    ohT = (gj == batc_ref[...]).astype(jnp.float32)
    cnt = jnp.maximum(jnp.sum(oh, axis=1, keepdims=True), 1.0)
    mean = jnp.dot(oh, h, preferred_element_type=jnp.float32) / cnt
    xc = h - gms_ref[...] * jnp.dot(ohT, mean,
                                    preferred_element_type=jnp.float32)
    var = jnp.dot(oh, xc * xc, preferred_element_type=jnp.float32) / cnt
    hn = gw_ref[...] * xc / jnp.sqrt(
        jnp.dot(ohT, var, preferred_element_type=jnp.float32) + 1e-5
    ) + gb_ref[...]
    hn = jnp.maximum(hn, 0.01 * hn)
    h2 = jnp.dot(hn, w_ref[...], preferred_element_type=jnp.float32)
    h_ref[...] = h2
    asv = jnp.sum(h2 * as_ref[...], axis=1, keepdims=True)
    adv = jnp.sum(h2 * ad_ref[...], axis=1, keepdims=True)
    asv_ref[...] = asv
    adv_ref[...] = adv
    m = jnp.max(asv) + jnp.max(adv)
    c_ref[...] = jnp.full((1, 1), 1.0, jnp.float32) * jnp.maximum(m, 0.2 * m)


def _tc_mid(parts, den0, den1, b, batr, batc, gw, gb, gms, w, a_s, a_d):
    return pl.pallas_call(
        _mid_body,
        out_shape=[
            jax.ShapeDtypeStruct((N, H), jnp.float32),
            jax.ShapeDtypeStruct((N, 1), jnp.float32),
            jax.ShapeDtypeStruct((N, 1), jnp.float32),
            jax.ShapeDtypeStruct((1, 1), jnp.float32),
        ],
    )(parts, den0, den1, b, batr, batc, gw, gb, gms, w, a_s, a_d)


def _fin_body(parts_ref, den0_ref, den1_ref, b_ref, batr_ref,
              hw1_ref, hb1_ref, hw2_ref, hb2_ref, out_ref):
    h = _combine(parts_ref, den0_ref, den1_ref, b_ref)
    gi = lax.broadcasted_iota(jnp.int32, (G, N), 0)
    oh = (gi == batr_ref[...]).astype(jnp.float32)
    pooled = jnp.dot(oh, h, preferred_element_type=jnp.float32)
    z = jnp.dot(pooled, hw1_ref[...],
                preferred_element_type=jnp.float32) + hb1_ref[...]
    z = jnp.maximum(z, 0.01 * z)
    out_ref[...] = jnp.dot(z, hw2_ref[...],
                           preferred_element_type=jnp.float32) + hb2_ref[...]


def _tc_fin(parts, den0, den1, b, batr, hw1, hb1, hw2, hb2):
    return pl.pallas_call(
        _fin_body,
        out_shape=jax.ShapeDtypeStruct((G, A), jnp.float32),
    )(parts, den0, den1, b, batr, hw1, hb1, hw2, hb2)


_MESH = plsc.VectorSubcoreMesh(core_axis_name="c", subcore_axis_name="s")


@functools.partial(
    pl.kernel,
    out_type=[
        jax.ShapeDtypeStruct((2, NP, H), jnp.float32),
        jax.ShapeDtypeStruct((2, NP), jnp.float32),
    ],
    mesh=_MESH,
    scratch_types=[
        pltpu.VMEM_SHARED((NP, H), jnp.float32),
        pltpu.VMEM_SHARED((NP,), jnp.float32),
        pltpu.VMEM_SHARED((N,), jnp.float32),
        pltpu.VMEM_SHARED((N,), jnp.float32),
        [pltpu.VMEM((EC,), jnp.int32)] * SB,
        [pltpu.VMEM((2, EC // 2), jnp.int32)] * SB,
        [pltpu.VMEM((EC,), jnp.float32)] * SB,
        [pltpu.VMEM((EC,), jnp.float32)] * SB,
        [pltpu.VMEM((2, EC // 2), jnp.float32)] * SB,
        [pltpu.VMEM((EC, H), jnp.float32)] * SB,
        pltpu.VMEM((EC,), jnp.float32),
        [pltpu.SemaphoreType.DMA] * SB,
        [pltpu.SemaphoreType.DMA] * SB,
        [pltpu.SemaphoreType.DMA] * SB,
        [[pltpu.SemaphoreType.DMA] * 2] * SB,
        [[pltpu.SemaphoreType.DMA] * 2] * SB,
    ],
)
def _sc_edge(h_hbm, asv_hbm, adv_hbm, src_hbm, dst2_hbm, c_hbm,
             out_hbm, den_hbm,
             out_sh, den_sh, asv_sh, adv_sh,
             srcv, dstv, asg, adg, pv, rows, cv,
             sem_r, sem_a, sem_b, sem_w1, sem_w2):
    c = lax.axis_index("c")
    s = lax.axis_index("s")
    wid = c * 16 + s

    zero16 = jnp.zeros((16,), jnp.float32)
    zrows = rows[0]

    def zbody(i, carry):
        for k in range(H // 16):
            zrows[i, pl.ds(k * 16, 16)] = zero16
        return carry

    lax.fori_loop(0, EC, zbody, 0)

    @pl.when(s == 0)
    def _():
        pltpu.sync_copy(asv_hbm, asv_sh)
        pltpu.sync_copy(adv_hbm, adv_sh)

    pltpu.sync_copy(c_hbm, cv)

    for t in range(5):
        idxr = s + 16 * t

        @pl.when(idxr < NRC)
        def _():
            pltpu.sync_copy(zrows, out_sh.at[pl.ds(idxr * EC, EC)])
            pltpu.sync_copy(zrows.at[0], den_sh.at[pl.ds(idxr * EC, EC)])

    plsc.subcore_barrier()
    cvec = cv[pl.ds(0, 16)]

    def cid_of(k):
        return wid + NW * k

    HEC = EC // 2

    def issue(k, b):
        base = cid_of(k) * EC
        pltpu.sync_copy(src_hbm.at[pl.ds(base, EC)], srcv[b])
        pltpu.async_copy(h_hbm.at[srcv[b]], rows[b], sem_r[b])
        pltpu.async_copy(asv_sh.at[srcv[b]], asg[b], sem_a[b])
        pltpu.sync_copy(dst2_hbm.at[pl.ds(cid_of(k) * 2, 2)], dstv[b])
        for j in range(2):
            pltpu.async_copy(adv_sh.at[dstv[b].at[j]],
                             adg[b].at[pl.ds(j * HEC, HEC)], sem_b[b])

    def consume(b):
        pltpu.make_async_copy(h_hbm.at[srcv[b]], rows[b], sem_r[b]).wait()
        pltpu.make_async_copy(asv_sh.at[srcv[b]], asg[b], sem_a[b]).wait()
        for j in range(2):
            pltpu.make_async_copy(adv_sh.at[dstv[b].at[j]],
                                  adg[b].at[pl.ds(j * HEC, HEC)],
                                  sem_b[b]).wait()
        for j2 in range(2):
            for jj in range(HEC // 16):
                sle = pl.ds(j2 * HEC + jj * 16, 16)
                e = asg[b][sle] + adg[b][sle]
                e = jnp.maximum(e, 0.2 * e)
                pv[b][j2, pl.ds(jj * 16, 16)] = jnp.exp(e - cvec)

        for j2 in range(2):
            @functools.partial(plsc.parallel_loop, 0, HEC // 16)
            def _(jj, j2=j2):
                pvec = pv[b][j2, pl.ds(jj * 16, 16)]
                for m in range(16):
                    psc = pvec[m]
                    i = j2 * HEC + jj * 16 + m
                    for k in range(H // 16):
                        slk = pl.ds(k * 16, 16)
                        rows[b][i, slk] = rows[b][i, slk] * psc
            pltpu.async_copy(rows[b].at[pl.ds(j2 * HEC, HEC)],
                             out_sh.at[dstv[b].at[j2]],
                             sem_w1[b][j2], add=True)
            pltpu.async_copy(pv[b].at[j2], den_sh.at[dstv[b].at[j2]],
                             sem_w2[b][j2], add=True)

    def wait_scatter(b):
        for j2 in range(2):
            pltpu.make_async_copy(rows[b].at[pl.ds(j2 * HEC, HEC)],
                                  out_sh.at[dstv[b].at[j2]],
                                  sem_w1[b][j2]).wait()
            pltpu.make_async_copy(pv[b].at[j2], den_sh.at[dstv[b].at[j2]],
                                  sem_w2[b][j2]).wait()

    def issue_steady(k, b):
        base = cid_of(k) * EC
        pltpu.sync_copy(src_hbm.at[pl.ds(base, EC)], srcv[b])
        pltpu.async_copy(asv_sh.at[srcv[b]], asg[b], sem_a[b])
        wait_scatter(b)
        pltpu.sync_copy(dst2_hbm.at[pl.ds(cid_of(k) * 2, 2)], dstv[b])
        pltpu.async_copy(h_hbm.at[srcv[b]], rows[b], sem_r[b])
        for j in range(2):
            pltpu.async_copy(adv_sh.at[dstv[b].at[j]],
                             adg[b].at[pl.ds(j * HEC, HEC)], sem_b[b])

    for b in range(SB):
        @pl.when(cid_of(b) < NCHUNK)
        def _():
            issue(b, b)

    def iter_body(t, carry):
        for b in range(SB):
            k = SB * t + b
            kn = k + SB

            @pl.when(cid_of(k) < NCHUNK)
            def _():
                consume(b)

            @pl.when(cid_of(kn) < NCHUNK)
            def _():
                issue_steady(kn, b)

        return carry

    lax.fori_loop(0, NITER, iter_body, 0)

    m_chunks = (NCHUNK - wid + NW - 1) // NW
    for b in range(SB):
        kb = jnp.where((m_chunks - 1) % SB == b, m_chunks - 1, m_chunks - 2)

        @pl.when(kb >= 0)
        def _():
            wait_scatter(b)

    plsc.subcore_barrier()

    for t in range(5):
        idxr = s + 16 * t

        @pl.when(idxr < NRC)
        def _():
            pltpu.sync_copy(out_sh.at[pl.ds(idxr * EC, EC)],
                            out_hbm.at[c, pl.ds(idxr * EC, EC)])
            pltpu.sync_copy(den_sh.at[pl.ds(idxr * EC, EC)],
                            den_hbm.at[c, pl.ds(idxr * EC, EC)])


def kernel(x, edge_index, batch, W1, as1, ad1, b1, W2, as2, ad2, b2,
           W3, as3, ad3, b3, gn1_w, gn1_b, gn1_ms, gn2_w, gn2_b, gn2_ms,
           hW1, hb1, hW2, hb2):
    src = edge_index[0]
    dst2 = edge_index[1].reshape(E // (EC // 2), EC // 2)
    batr = batch.reshape(1, N)
    batc = batch.reshape(N, 1)

    def edge(h, asv, adv, c1):
        c128 = jnp.broadcast_to(c1.reshape(()), (EC,))
        parts, den = _sc_edge(h, asv.reshape(N), adv.reshape(N),
                              src, dst2, c128)
        return (parts[:, :N, :], den[0, :N].reshape(N, 1),
                den[1, :N].reshape(N, 1))

    h1, asv1, adv1, c1 = _tc_pre(x, W1, as1.reshape(1, H), ad1.reshape(1, H))
    parts1, d10, d11 = edge(h1, asv1, adv1, c1)
    h2, asv2, adv2, c2 = _tc_mid(parts1, d10, d11, b1.reshape(1, H),
                                 batr, batc, gn1_w.reshape(1, H),
                                 gn1_b.reshape(1, H), gn1_ms.reshape(1, H),
                                 W2, as2.reshape(1, H), ad2.reshape(1, H))
    parts2, d20, d21 = edge(h2, asv2, adv2, c2)
    h3, asv3, adv3, c3 = _tc_mid(parts2, d20, d21, b2.reshape(1, H),
                                 batr, batc, gn2_w.reshape(1, H),
                                 gn2_b.reshape(1, H), gn2_ms.reshape(1, H),
                                 W3, as3.reshape(1, H), ad3.reshape(1, H))
    parts3, d30, d31 = edge(h3, asv3, adv3, c3)
    return _tc_fin(parts3, d30, d31, b3.reshape(1, H), batr,
                   hW1, hb1.reshape(1, H), hW2, hb2.reshape(1, A))
